# baseline (speedup 1.0000x reference)
import functools

import jax
import jax.numpy as jnp
from jax import lax
from jax.experimental import pallas as pl
from jax.experimental.pallas import tpu as pltpu
from jax.experimental.pallas import tpu_sc as plsc

_B, _N, _C = 2, 2000, 81
_ROW = 4 * _C
_ROWS = _B * _N
_CHUNK_ROWS = 16
_CHUNK = _CHUNK_ROWS * _ROW
_LCHUNK = _CHUNK_ROWS * _C
_NCHUNKS = _ROWS // _CHUNK_ROWS
_NC, _NS, _L = 2, 16, 16
_NW = _NC * _NS
_GROUPS = _LCHUNK // _L
_EPS_SUM = 0.0001 * (_ROWS * 4 * (_C - 1))


@functools.partial(
    pl.kernel,
    out_type=jax.ShapeDtypeStruct((_NW, _L), jnp.float32),
    mesh=plsc.VectorSubcoreMesh(core_axis_name="c", subcore_axis_name="s"),
    scratch_types=[
        pltpu.VMEM((_CHUNK,), jnp.float32),
        pltpu.VMEM((_CHUNK,), jnp.float32),
        pltpu.VMEM((_LCHUNK,), jnp.float32),
        pltpu.VMEM((_L,), jnp.float32),
    ],
)
def _sc_loss(out_hbm, tgt_hbm, lbl_hbm, res_hbm, o_v, t_v, l_v, r_v):
    wid = lax.axis_index("s") * _NC + lax.axis_index("c")
    nk = (_NCHUNKS - 1 - wid) // _NW + 1
    lane = lax.iota(jnp.int32, _L)
    lane4 = lane >> 2
    eidx = [lane4 + 4 * j for j in range(4)]
    gd = lax.GatherDimensionNumbers(
        offset_dims=(), collapsed_slice_dims=(0,), start_index_map=(0,)
    )

    def expand(lv, j):
        return lax.gather(
            lv, eidx[j][:, None], gd, slice_sizes=(1,),
            mode=lax.GatherScatterMode.PROMISE_IN_BOUNDS,
        )

    def chunk_body(k, carry):
        acc0, cnt0 = carry
        c = wid + k * _NW
        pltpu.sync_copy(out_hbm.at[pl.ds(c * _CHUNK, _CHUNK)], o_v)
        pltpu.sync_copy(tgt_hbm.at[pl.ds(c * _CHUNK, _CHUNK)], t_v)
        pltpu.sync_copy(lbl_hbm.at[pl.ds(c * _LCHUNK, _LCHUNK)], l_v)

        def grp_body(g, vc):
            acc, cnt, r = vc
            lv = l_v[pl.ds(g * _L, _L)]
            rv0 = lane4 + r
            for j in range(4):
                o = o_v[pl.ds(g * 64 + j * _L, _L)]
                t = t_v[pl.ds(g * 64 + j * _L, _L)]
                lbl = expand(lv, j)
                rv = rv0 + 4 * j
                keep = jnp.where(rv * (rv - 81) == 0, 0.0, 1.0)
                pos = jnp.where(lbl == 1.0, 1.0, 0.0)
                om = o * pos
                x = jnp.abs(om - t)
                sl = jnp.where(x < 1.0, (0.5 * x) * x, x - 0.5)
                acc = acc + sl * keep
                cnt = cnt + pos * keep
            r = r + _L
            r = jnp.where(r >= 81, r - 81, r)
            return acc, cnt, r

        acc0, cnt0, _ = lax.fori_loop(
            0, _GROUPS, grp_body, (acc0, cnt0, jnp.int32(0))
        )
        return acc0, cnt0

    zero = jnp.zeros((_L,), jnp.float32)
    acc, cnt = lax.fori_loop(0, nk, chunk_body, (zero, zero))

    def lanesum(x):
        for sh in (8, 4, 2, 1):
            x = x + lax.gather(
                x, (lane ^ sh)[:, None], gd, slice_sizes=(1,),
                mode=lax.GatherScatterMode.PROMISE_IN_BOUNDS,
            )
        return x

    loss_s = lanesum(acc)
    cnt_s = lanesum(cnt)
    is0 = jnp.where(lane == 0, 1.0, 0.0)
    is1 = jnp.where(lane == 1, 1.0, 0.0)
    r_v[...] = loss_s * is0 + cnt_s * is1
    pltpu.sync_copy(r_v, res_hbm.at[wid])


@jax.jit
def kernel(output, target, labels_target):
    part = _sc_loss(
        output.reshape(-1), target.reshape(-1), labels_target.reshape(-1)
    )
    loss_sum = jnp.sum(part[:, 0])
    cnt = jnp.sum(part[:, 1])
    return loss_sum / (jnp.float32(_EPS_SUM) + cnt)


# speedup vs baseline: 1.7502x; 1.7502x over previous
import functools

import jax
import jax.numpy as jnp
from jax import lax
from jax.experimental import pallas as pl
from jax.experimental.pallas import tpu as pltpu
from jax.experimental.pallas import tpu_sc as plsc

_B, _N, _C = 2, 2000, 81
_ROW = 4 * _C
_ROWS = _B * _N
_CHUNK_ROWS = 16
_NCHUNKS = _ROWS // _CHUNK_ROWS
_NC, _NS, _L = 2, 16, 16
_NW = _NC * _NS
_KMAX = -(-_NCHUNKS // _NW)
_NV = _ROW // _L
_TAIL = _ROW - _L
_EPS_SUM = 0.0001 * (_ROWS * 4 * (_C - 1))


@functools.partial(
    pl.kernel,
    out_type=jax.ShapeDtypeStruct((_NW, _L), jnp.float32),
    mesh=plsc.VectorSubcoreMesh(core_axis_name="c", subcore_axis_name="s"),
    scratch_types=[
        pltpu.VMEM((2 * _CHUNK_ROWS, _ROW), jnp.float32),
        pltpu.VMEM((2 * _CHUNK_ROWS, _ROW), jnp.float32),
        pltpu.VMEM((2 * _CHUNK_ROWS, _C), jnp.float32),
        pltpu.VMEM((_L,), jnp.float32),
        pltpu.SemaphoreType.DMA((2,)),
    ],
)
def _sc_loss(out_hbm, tgt_hbm, lbl_hbm, res_hbm, o2, t2, l2, r_v, sems):
    wid = lax.axis_index("s") * _NC + lax.axis_index("c")
    lane = lax.iota(jnp.int32, _L)
    lane4 = lane >> 2
    eidx = [lane4 + 4 * j for j in range(4)]
    tidx = lane4 + 12
    gd = lax.GatherDimensionNumbers(
        offset_dims=(), collapsed_slice_dims=(0,), start_index_map=(0,)
    )
    m_lt4 = lane < 4
    m_lt12 = lane < 12
    keep0 = jnp.where(lane == 0, 0.0, 1.0)
    c15 = jnp.where(lane == 15, 1.0, 0.0)

    def xlane(v, idx):
        return lax.gather(
            v, idx[:, None], gd, slice_sizes=(1,),
            mode=lax.GatherScatterMode.PROMISE_IN_BOUNDS,
        )

    def copies(k, slot):
        ce = jnp.minimum(wid + k * _NW, _NCHUNKS - 1)
        r0 = ce * _CHUNK_ROWS
        b0 = slot * _CHUNK_ROWS
        return (
            (out_hbm.at[pl.ds(r0, _CHUNK_ROWS), :],
             o2.at[pl.ds(b0, _CHUNK_ROWS), :]),
            (tgt_hbm.at[pl.ds(r0, _CHUNK_ROWS), :],
             t2.at[pl.ds(b0, _CHUNK_ROWS), :]),
            (lbl_hbm.at[pl.ds(r0, _CHUNK_ROWS), :],
             l2.at[pl.ds(b0, _CHUNK_ROWS), :]),
        )

    def issue(k, slot):
        for src, dst in copies(k, slot):
            pltpu.async_copy(src, dst, sems.at[slot])

    def drain(k, slot):
        for src, dst in copies(k, slot):
            pltpu.make_async_copy(src, dst, sems.at[slot]).wait()

    zero = jnp.zeros((_L,), jnp.float32)
    issue(0, 0)

    def chunk_body(k, carry):
        slot = k & 1

        @pl.when(k < _KMAX - 1)
        def _():
            issue(k + 1, slot ^ 1)

        drain(k, slot)

        def row_body(r, vc):
            a0, a1, a2, a3, ccnt = vc
            rr = slot * _CHUNK_ROWS + r
            pkl = []
            for q in range(5):
                lv = l2[rr, pl.ds(q * _L, _L)]
                pk_q = jnp.where(lv == 1.0, keep0 if q == 0 else 1.0, 0.0)
                ccnt = ccnt + pk_q
                pkl.append(pk_q)
            lv5 = l2[rr, pl.ds(_C - _L, _L)]
            pk5 = jnp.where(lv5 == 1.0, 1.0, 0.0)
            ccnt = ccnt + jnp.where(lv5 == 1.0, c15, 0.0)
            accs = [a0, a1, a2, a3]
            for v in range(_NV):
                o = o2[rr, pl.ds(v * _L, _L)]
                t = t2[rr, pl.ds(v * _L, _L)]
                pk = xlane(pkl[v >> 2], eidx[v & 3])
                x = jnp.abs(o * pk - t)
                u = jnp.minimum(x, 1.0)
                pq = u * ((x + x) - u)
                if v == 0:
                    pq = jnp.where(m_lt4, 0.0, pq)
                accs[v & 3] = accs[v & 3] + pq
            o = o2[rr, pl.ds(_TAIL, _L)]
            t = t2[rr, pl.ds(_TAIL, _L)]
            pk = xlane(pk5, tidx)
            x = jnp.abs(o * pk - t)
            u = jnp.minimum(x, 1.0)
            pq = u * ((x + x) - u)
            accs[0] = accs[0] + jnp.where(m_lt12, 0.0, pq)
            return accs[0], accs[1], accs[2], accs[3], ccnt

        b0, b1, b2c, b3, ccnt = lax.fori_loop(
            0, _CHUNK_ROWS, row_body, (zero, zero, zero, zero, zero)
        )
        acc, cnt = carry
        wf = jnp.where(wid + k * _NW < _NCHUNKS, 1.0, 0.0)
        acc = acc + ((b0 + b1) + (b2c + b3)) * wf
        cnt = cnt + ccnt * wf
        return acc, cnt

    acc, cnt = lax.fori_loop(0, _KMAX, chunk_body, (zero, zero))

    def lanesum(x):
        for sh in (8, 4, 2, 1):
            x = x + xlane(x, lane ^ sh)
        return x

    loss_s = lanesum(acc)
    cnt_s = lanesum(cnt)
    is0 = jnp.where(lane == 0, 1.0, 0.0)
    is1 = jnp.where(lane == 1, 1.0, 0.0)
    r_v[...] = loss_s * is0 + cnt_s * is1
    pltpu.sync_copy(r_v, res_hbm.at[wid])


@jax.jit
def kernel(output, target, labels_target):
    part = _sc_loss(
        output.reshape(_ROWS, _ROW),
        target.reshape(_ROWS, _ROW),
        labels_target.reshape(_ROWS, _C),
    )
    loss_sum = 0.5 * jnp.sum(part[:, 0])
    cnt = jnp.sum(part[:, 1])
    return loss_sum / (jnp.float32(_EPS_SUM) + 4.0 * cnt)


# speedup vs baseline: 1.7518x; 1.0009x over previous
import functools

import jax
import jax.numpy as jnp
from jax import lax
from jax.experimental import pallas as pl
from jax.experimental.pallas import tpu as pltpu
from jax.experimental.pallas import tpu_sc as plsc

_B, _N, _C = 2, 2000, 81
_ROW = 4 * _C
_ROWS = _B * _N
_CHUNK_ROWS = 16
_NCHUNKS = _ROWS // _CHUNK_ROWS
_NC, _NS, _L = 2, 16, 16
_NW = _NC * _NS
_KMAX = -(-_NCHUNKS // _NW)
_NV = _ROW // _L
_TAIL = _ROW - _L
_EPS_SUM = 0.0001 * (_ROWS * 4 * (_C - 1))


@functools.partial(
    pl.kernel,
    out_type=jax.ShapeDtypeStruct((_NW, _L), jnp.float32),
    mesh=plsc.VectorSubcoreMesh(core_axis_name="c", subcore_axis_name="s"),
    scratch_types=[
        pltpu.VMEM((2 * _CHUNK_ROWS, _ROW), jnp.float32),
        pltpu.VMEM((2 * _CHUNK_ROWS, _ROW), jnp.float32),
        pltpu.VMEM((2 * _CHUNK_ROWS, _C), jnp.float32),
        pltpu.VMEM((_L,), jnp.float32),
        pltpu.SemaphoreType.DMA((2,)),
    ],
    compiler_params=pltpu.CompilerParams(use_tc_tiling_on_sc=True),
)
def _sc_loss(out_hbm, tgt_hbm, lbl_hbm, res_hbm, o2, t2, l2, r_v, sems):
    wid = lax.axis_index("s") * _NC + lax.axis_index("c")
    lane = lax.iota(jnp.int32, _L)
    lane4 = lane >> 2
    eidx = [lane4 + 4 * j for j in range(4)]
    tidx = lane4 + 12
    gd = lax.GatherDimensionNumbers(
        offset_dims=(), collapsed_slice_dims=(0,), start_index_map=(0,)
    )
    m_lt4 = lane < 4
    m_lt12 = lane < 12
    keep0 = jnp.where(lane == 0, 0.0, 1.0)
    c15 = jnp.where(lane == 15, 1.0, 0.0)

    def xlane(v, idx):
        return lax.gather(
            v, idx[:, None], gd, slice_sizes=(1,),
            mode=lax.GatherScatterMode.PROMISE_IN_BOUNDS,
        )

    def copies(k, slot):
        ce = jnp.minimum(wid + k * _NW, _NCHUNKS - 1)
        r0 = ce * _CHUNK_ROWS
        b0 = slot * _CHUNK_ROWS
        return (
            (out_hbm.at[pl.ds(r0, _CHUNK_ROWS), :],
             o2.at[pl.ds(b0, _CHUNK_ROWS), :]),
            (tgt_hbm.at[pl.ds(r0, _CHUNK_ROWS), :],
             t2.at[pl.ds(b0, _CHUNK_ROWS), :]),
            (lbl_hbm.at[pl.ds(r0, _CHUNK_ROWS), :],
             l2.at[pl.ds(b0, _CHUNK_ROWS), :]),
        )

    def issue(k, slot):
        for src, dst in copies(k, slot):
            pltpu.async_copy(src, dst, sems.at[slot])

    def drain(k, slot):
        for src, dst in copies(k, slot):
            pltpu.make_async_copy(src, dst, sems.at[slot]).wait()

    zero = jnp.zeros((_L,), jnp.float32)
    issue(0, 0)

    def chunk_body(k, carry):
        slot = k & 1

        @pl.when(k < _KMAX - 1)
        def _():
            issue(k + 1, slot ^ 1)

        drain(k, slot)

        def row_body(r, vc):
            a0, a1, a2, a3, ccnt = vc
            rr = slot * _CHUNK_ROWS + r
            pkl = []
            for q in range(5):
                lv = l2[rr, pl.ds(q * _L, _L)]
                pk_q = jnp.where(lv == 1.0, keep0 if q == 0 else 1.0, 0.0)
                ccnt = ccnt + pk_q
                pkl.append(pk_q)
            lv5 = l2[rr, pl.ds(_C - _L, _L)]
            pk5 = jnp.where(lv5 == 1.0, 1.0, 0.0)
            ccnt = ccnt + jnp.where(lv5 == 1.0, c15, 0.0)
            accs = [a0, a1, a2, a3]
            for v in range(_NV):
                o = o2[rr, pl.ds(v * _L, _L)]
                t = t2[rr, pl.ds(v * _L, _L)]
                pk = xlane(pkl[v >> 2], eidx[v & 3])
                x = jnp.abs(o * pk - t)
                u = jnp.minimum(x, 1.0)
                pq = u * ((x + x) - u)
                if v == 0:
                    pq = jnp.where(m_lt4, 0.0, pq)
                accs[v & 3] = accs[v & 3] + pq
            o = o2[rr, pl.ds(_TAIL, _L)]
            t = t2[rr, pl.ds(_TAIL, _L)]
            pk = xlane(pk5, tidx)
            x = jnp.abs(o * pk - t)
            u = jnp.minimum(x, 1.0)
            pq = u * ((x + x) - u)
            accs[0] = accs[0] + jnp.where(m_lt12, 0.0, pq)
            return accs[0], accs[1], accs[2], accs[3], ccnt

        b0, b1, b2c, b3, ccnt = lax.fori_loop(
            0, _CHUNK_ROWS, row_body, (zero, zero, zero, zero, zero)
        )
        acc, cnt = carry
        wf = jnp.where(wid + k * _NW < _NCHUNKS, 1.0, 0.0)
        acc = acc + ((b0 + b1) + (b2c + b3)) * wf
        cnt = cnt + ccnt * wf
        return acc, cnt

    acc, cnt = lax.fori_loop(0, _KMAX, chunk_body, (zero, zero))

    def lanesum(x):
        for sh in (8, 4, 2, 1):
            x = x + xlane(x, lane ^ sh)
        return x

    loss_s = lanesum(acc)
    cnt_s = lanesum(cnt)
    is0 = jnp.where(lane == 0, 1.0, 0.0)
    is1 = jnp.where(lane == 1, 1.0, 0.0)
    r_v[...] = loss_s * is0 + cnt_s * is1
    pltpu.sync_copy(r_v, res_hbm.at[wid])


@jax.jit
def kernel(output, target, labels_target):
    part = _sc_loss(
        output.reshape(_ROWS, _ROW),
        target.reshape(_ROWS, _ROW),
        labels_target.reshape(_ROWS, _C),
    )
    loss_sum = 0.5 * jnp.sum(part[:, 0])
    cnt = jnp.sum(part[:, 1])
    return loss_sum / (jnp.float32(_EPS_SUM) + 4.0 * cnt)


# speedup vs baseline: 2.7061x; 1.5448x over previous
import functools

import jax
import jax.numpy as jnp
from jax import lax
from jax.experimental import pallas as pl
from jax.experimental.pallas import tpu as pltpu
from jax.experimental.pallas import tpu_sc as plsc

_B, _N, _C = 2, 2000, 81
_NC, _NS, _L = 2, 16, 16
_NW = _NC * _NS
_NG = _C - 1
_KMAX = -(-_NG // _NW)
_NV = _N // _L
_EPS_SUM = 0.0001 * (_B * _N * 4 * (_C - 1))


@functools.partial(
    pl.kernel,
    out_type=jax.ShapeDtypeStruct((_NW, _L), jnp.float32),
    mesh=plsc.VectorSubcoreMesh(core_axis_name="c", subcore_axis_name="s"),
    scratch_types=[
        pltpu.VMEM((4, _B, _N), jnp.float32),
        pltpu.VMEM((4, _B, _N), jnp.float32),
        pltpu.VMEM((_B, _N), jnp.float32),
        pltpu.VMEM((4, _B, _N), jnp.float32),
        pltpu.VMEM((4, _B, _N), jnp.float32),
        pltpu.VMEM((_B, _N), jnp.float32),
        pltpu.VMEM((_L,), jnp.float32),
        pltpu.SemaphoreType.DMA,
        pltpu.SemaphoreType.DMA,
    ],
)
def _sc_loss(out_hbm, tgt_hbm, lbl_hbm, res_hbm,
             o_a, t_a, l_a, o_b, t_b, l_b, r_v, sem_a, sem_b):
    wid = lax.axis_index("s") * _NC + lax.axis_index("c")
    lane = lax.iota(jnp.int32, _L)
    gd = lax.GatherDimensionNumbers(
        offset_dims=(), collapsed_slice_dims=(0,), start_index_map=(0,)
    )

    bufs = [(o_a, t_a, l_a, sem_a), (o_b, t_b, l_b, sem_b)]

    def copies(k, slot):
        g = jnp.minimum(1 + wid + k * _NW, _C - 1)
        o_v, t_v, l_v, sem = bufs[slot]
        return (
            (out_hbm.at[pl.ds(4 * g, 4), :, :], o_v, sem),
            (tgt_hbm.at[pl.ds(4 * g, 4), :, :], t_v, sem),
            (lbl_hbm.at[g], l_v, sem),
        )

    def issue(k, slot):
        for src, dst, sem in copies(k, slot):
            pltpu.async_copy(src, dst, sem)

    def drain(k, slot):
        for src, dst, sem in copies(k, slot):
            pltpu.make_async_copy(src, dst, sem).wait()

    zero = jnp.zeros((_L,), jnp.float32)

    def process(slot):
        o_v, t_v, l_v, _ = bufs[slot]

        def vec_body(v, vc):
            accs = list(vc[:8])
            cnt = vc[8]
            n0 = v * _L
            for b in range(_B):
                lb = l_v[b, pl.ds(n0, _L)]
                pos = jnp.where(lb == 1.0, 1.0, 0.0)
                cnt = cnt + pos
                for j in range(4):
                    o = o_v[j, b, pl.ds(n0, _L)]
                    t = t_v[j, b, pl.ds(n0, _L)]
                    x = jnp.abs(o * pos - t)
                    u = jnp.minimum(x, 1.0)
                    i = 4 * b + j
                    accs[i] = accs[i] + u * ((x + x) - u)
            return (*accs, cnt)

        out = lax.fori_loop(0, _NV, vec_body, (zero,) * 9)
        a = out[0]
        for i in range(1, 8):
            a = a + out[i]
        return a, out[8]

    acc = zero
    cnt = zero
    issue(0, 0)
    for k in range(_KMAX):
        slot = k & 1
        if k + 1 < _KMAX:
            issue(k + 1, slot ^ 1)
        drain(k, slot)
        cacc, ccnt = process(slot)
        wf = jnp.where(1 + wid + k * _NW < _C, 1.0, 0.0)
        acc = acc + cacc * wf
        cnt = cnt + ccnt * wf

    def lanesum(x):
        for sh in (8, 4, 2, 1):
            x = x + lax.gather(
                x, (lane ^ sh)[:, None], gd, slice_sizes=(1,),
                mode=lax.GatherScatterMode.PROMISE_IN_BOUNDS,
            )
        return x

    loss_s = lanesum(acc)
    cnt_s = lanesum(cnt)
    is0 = jnp.where(lane == 0, 1.0, 0.0)
    is1 = jnp.where(lane == 1, 1.0, 0.0)
    r_v[...] = loss_s * is0 + cnt_s * is1
    pltpu.sync_copy(r_v, res_hbm.at[wid])


@jax.jit
def kernel(output, target, labels_target):
    part = _sc_loss(
        jnp.transpose(output, (2, 0, 1)),
        jnp.transpose(target, (2, 0, 1)),
        jnp.transpose(labels_target, (2, 0, 1)),
    )
    loss_sum = 0.5 * jnp.sum(part[:, 0])
    cnt = jnp.sum(part[:, 1])
    return loss_sum / (jnp.float32(_EPS_SUM) + 4.0 * cnt)


# speedup vs baseline: 2.8474x; 1.0522x over previous
import functools

import jax
import jax.numpy as jnp
from jax import lax
from jax.experimental import pallas as pl
from jax.experimental.pallas import tpu as pltpu
from jax.experimental.pallas import tpu_sc as plsc

_B, _N, _C = 2, 2000, 81
_NC, _NS, _L = 2, 16, 16
_NW = _NC * _NS
_NU = 2 * (_C - 1)
_KMAX = _NU // _NW
_NV = _N // _L
_EPS_SUM = 0.0001 * (_B * _N * 4 * (_C - 1))


@functools.partial(
    pl.kernel,
    out_type=jax.ShapeDtypeStruct((_NW, _L), jnp.float32),
    mesh=plsc.VectorSubcoreMesh(core_axis_name="c", subcore_axis_name="s"),
    scratch_types=[
        pltpu.VMEM((2, _B, _N), jnp.float32),
        pltpu.VMEM((2, _B, _N), jnp.float32),
        pltpu.VMEM((_B, _N), jnp.float32),
        pltpu.VMEM((2, _B, _N), jnp.float32),
        pltpu.VMEM((2, _B, _N), jnp.float32),
        pltpu.VMEM((_B, _N), jnp.float32),
        pltpu.VMEM((_L,), jnp.float32),
        pltpu.SemaphoreType.DMA,
        pltpu.SemaphoreType.DMA,
    ],
)
def _sc_loss(out_hbm, tgt_hbm, lbl_hbm, res_hbm,
             o_a, t_a, l_a, o_b, t_b, l_b, r_v, sem_a, sem_b):
    wid = lax.axis_index("s") * _NC + lax.axis_index("c")
    lane = lax.iota(jnp.int32, _L)
    gd = lax.GatherDimensionNumbers(
        offset_dims=(), collapsed_slice_dims=(0,), start_index_map=(0,)
    )

    bufs = [(o_a, t_a, l_a, sem_a), (o_b, t_b, l_b, sem_b)]

    def copies(k, slot):
        u = wid + k * _NW
        o_v, t_v, l_v, sem = bufs[slot]
        return (
            (out_hbm.at[pl.ds(4 + 2 * u, 2), :, :], o_v, sem),
            (tgt_hbm.at[pl.ds(4 + 2 * u, 2), :, :], t_v, sem),
            (lbl_hbm.at[1 + (u >> 1)], l_v, sem),
        )

    def issue(k, slot):
        for src, dst, sem in copies(k, slot):
            pltpu.async_copy(src, dst, sem)

    def drain(k, slot):
        for src, dst, sem in copies(k, slot):
            pltpu.make_async_copy(src, dst, sem).wait()

    zero = jnp.zeros((_L,), jnp.float32)

    def process(slot, carry):
        o_v, t_v, l_v, _ = bufs[slot]

        def vec_body(v, vc):
            accs = list(vc[:4])
            cnt = vc[4]
            n0 = v * _L
            for b in range(_B):
                lb = l_v[b, pl.ds(n0, _L)]
                pos = jnp.where(lb == 1.0, 1.0, 0.0)
                cnt = cnt + pos
                for j in range(2):
                    o = o_v[j, b, pl.ds(n0, _L)]
                    t = t_v[j, b, pl.ds(n0, _L)]
                    x = jnp.abs(o * pos - t)
                    u = jnp.minimum(x, 1.0)
                    i = 2 * b + j
                    accs[i] = accs[i] + u * ((x + x) - u)
            return (*accs, cnt)

        return lax.fori_loop(0, _NV, vec_body, carry, unroll=2)

    issue(0, 0)
    carry = (zero,) * 5
    for k in range(_KMAX):
        slot = k & 1
        if k + 1 < _KMAX:
            issue(k + 1, slot ^ 1)
        drain(k, slot)
        carry = process(slot, carry)

    acc = (carry[0] + carry[1]) + (carry[2] + carry[3])
    cnt = carry[4]

    def lanesum(x):
        for sh in (8, 4, 2, 1):
            x = x + lax.gather(
                x, (lane ^ sh)[:, None], gd, slice_sizes=(1,),
                mode=lax.GatherScatterMode.PROMISE_IN_BOUNDS,
            )
        return x

    loss_s = lanesum(acc)
    cnt_s = lanesum(cnt)
    is0 = jnp.where(lane == 0, 1.0, 0.0)
    is1 = jnp.where(lane == 1, 1.0, 0.0)
    r_v[...] = loss_s * is0 + cnt_s * is1
    pltpu.sync_copy(r_v, res_hbm.at[wid])


@jax.jit
def kernel(output, target, labels_target):
    part = _sc_loss(
        jnp.transpose(output, (2, 0, 1)),
        jnp.transpose(target, (2, 0, 1)),
        jnp.transpose(labels_target, (2, 0, 1)),
    )
    loss_sum = 0.5 * jnp.sum(part[:, 0])
    cnt2 = jnp.sum(part[:, 1])
    return loss_sum / (jnp.float32(_EPS_SUM) + 2.0 * cnt2)
